# baseline (speedup 1.0000x reference)
import functools

import jax
import jax.numpy as jnp
from jax import lax
from jax.experimental import pallas as pl
from jax.experimental.pallas import tpu as pltpu
from jax.experimental.pallas import tpu_sc as plsc

H = 2
OUT_M = 32
OUT_X = 32
NC = 2
NS = 16
NW = NC * NS
LANES = 16

E = 640000
EW = E // NW
EG = 160000
EGW = EG // NW
NG = 10000
NGP = 10240
C = 80
R = 2048
SH = 11
WA = 1000
WB = 512
CAP = E + C * WB


def _leaky(x):
    return jnp.where(x >= 0, x, 0.01 * x)


def _mesh():
    return plsc.VectorSubcoreMesh(core_axis_name="c", subcore_axis_name="s",
                                  num_cores=NC, num_subcores=NS)


def _sc_gather(tabs, idxs, win=200):
    e = idxs[0].shape[0]
    d = tabs[0].shape[1]
    e_pw = e // NW
    assert e_pw * NW == e and e_pw % win == 0
    wins = e_pw // win
    nt = len(tabs)

    @functools.partial(
        pl.kernel, mesh=_mesh(),
        out_type=jax.ShapeDtypeStruct((e, d), jnp.float32),
        scratch_types=[pltpu.VMEM((win,), jnp.int32) for _ in range(nt)]
        + [pltpu.VMEM((win, d), jnp.float32), pltpu.SemaphoreType.DMA],
    )
    def k(*refs):
        tab_refs = refs[:nt]
        idx_refs = refs[nt:2 * nt]
        out = refs[2 * nt]
        iv = refs[2 * nt + 1:2 * nt + 1 + nt]
        buf = refs[2 * nt + 1 + nt]
        sem = refs[2 * nt + 2 + nt]
        wid = lax.axis_index("s") * NC + lax.axis_index("c")
        base = wid * e_pw

        def body(i, carry):
            start = base + i * win
            for j in range(nt):
                pltpu.sync_copy(idx_refs[j].at[pl.ds(start, win)], iv[j])
            for j in range(nt):
                pltpu.async_copy(tab_refs[j].at[iv[j]], buf, sem,
                                 add=(j > 0)).wait()
            pltpu.sync_copy(buf, out.at[pl.ds(start, win)])
            return carry

        lax.fori_loop(0, wins, body, 0)

    return k(*tabs, *idxs)


def _row_call(body, n, br, in_arrs, in_blocks, out_shapes, out_blocks):
    return pl.pallas_call(
        body,
        grid=(n // br,),
        in_specs=[pl.BlockSpec(bs, im) for bs, im in in_blocks],
        out_specs=[pl.BlockSpec(bs, im) for bs, im in out_blocks],
        out_shape=out_shapes,
    )(*in_arrs)


def kernel(l_feats, m_feats, x_feats, graph_edge_index, lgraph_edge_index,
           W_lg_node, b_lg_node, W_lg_ni, W_lg_fij, W_lg_nj, lg_attn, bias_lg,
           W_g_node, b_g_node, W_g_ni, W_g_fij, W_g_nj, g_attn, bias_g):
    n_lg = m_feats.shape[0]
    in_l = l_feats.shape[1]
    hx = H * OUT_X
    g_src = graph_edge_index[0]
    g_dst = graph_edge_index[1]
    lg_src = lgraph_edge_index[0]
    lg_dst = lgraph_edge_index[1]

    f32 = jnp.float32

    w_m = jnp.concatenate(
        [W_lg_ni.T, W_lg_nj.T, W_lg_node[:, :in_l].T,
         W_g_fij.T, W_g_node[:, in_l:].T], axis=1)
    b_m = jnp.concatenate(
        [jnp.zeros((2 * hx,), f32), b_lg_node,
         jnp.zeros((hx,), f32), b_g_node])

    def m_body(x_ref, w_ref, b_ref, ni0_ref, njp_ref, hm1_ref, fgm_ref):
        acc = (jnp.dot(x_ref[...], w_ref[...],
                       preferred_element_type=f32) + b_ref[...])
        z = jnp.zeros((acc.shape[0], hx), f32)
        ni0_ref[...] = jnp.concatenate([acc[:, 0:hx], z], axis=1)
        njp_ref[...] = jnp.concatenate([z, acc[:, hx:2 * hx]], axis=1)
        hm1_ref[...] = acc[:, 2 * hx:3 * hx]
        fgm_ref[...] = acc[:, 3 * hx:5 * hx]

    br = 2000
    ni0, njp, hm1, fgm = _row_call(
        m_body, n_lg, br, [m_feats, w_m, b_m.reshape(1, 5 * hx)],
        [((br, in_l), lambda i: (i, 0)), ((in_l, 5 * hx), lambda i: (0, 0)),
         ((1, 5 * hx), lambda i: (0, 0))],
        [jax.ShapeDtypeStruct((n_lg, 2 * hx), f32),
         jax.ShapeDtypeStruct((n_lg, 2 * hx), f32),
         jax.ShapeDtypeStruct((n_lg, hx), f32),
         jax.ShapeDtypeStruct((n_lg, 2 * hx), f32)],
        [((br, 2 * hx), lambda i: (i, 0)), ((br, 2 * hx), lambda i: (i, 0)),
         ((br, hx), lambda i: (i, 0)), ((br, 2 * hx), lambda i: (i, 0))])

    def x_body(x_ref, w_ref, o_ref):
        o_ref[...] = jnp.dot(x_ref[...], w_ref[...],
                             preferred_element_type=f32)

    w_x = jnp.concatenate([W_lg_fij.T, W_lg_node[:, in_l:].T], axis=1)
    (pxy,) = _row_call(
        x_body, E, br, [x_feats, w_x],
        [((br, in_l), lambda i: (i, 0)), ((in_l, 2 * hx), lambda i: (0, 0))],
        [jax.ShapeDtypeStruct((E, 2 * hx), f32)],
        [((br, 2 * hx), lambda i: (i, 0))])

    q_t = W_g_node[:, :in_l].T
    w_l = jnp.concatenate([W_g_ni.T, q_t, W_g_nj.T, q_t], axis=1)

    def l_body(x_ref, w_ref, ps_ref, pd_ref):
        acc = jnp.dot(x_ref[...], w_ref[...], preferred_element_type=f32)
        ps_ref[...] = acc[:, :2 * hx]
        pd_ref[...] = acc[:, 2 * hx:]

    ps, pd = _row_call(
        l_body, NG, br, [l_feats, w_l],
        [((br, in_l), lambda i: (i, 0)), ((in_l, 4 * hx), lambda i: (0, 0))],
        [jax.ShapeDtypeStruct((NG, 2 * hx), f32),
         jax.ShapeDtypeStruct((NG, 2 * hx), f32)],
        [((br, 2 * hx), lambda i: (i, 0)), ((br, 2 * hx), lambda i: (i, 0))])

    a1b = _sc_gather([ni0, njp], [lg_src, lg_dst])
    agp = _sc_gather([ps, pd], [g_src, g_dst])

    def ex_body(a_ref, f_ref, attn_ref, bias_ref, o_ref):
        pre = a_ref[:, :hx] + a_ref[:, hx:] + f_ref[:, :hx] + bias_ref[...]
        xh = _leaky(pre).reshape(-1, H, OUT_X)
        attn = attn_ref[...].reshape(1, H, OUT_X)
        o_ref[...] = jnp.exp(jnp.sum(xh * attn, axis=-1))

    ex_in_specs = [
        ((br, 2 * hx), lambda i: (i, 0)), ((br, 2 * hx), lambda i: (i, 0)),
        ((1, hx), lambda i: (0, 0)), ((1, hx), lambda i: (0, 0))]
    (ex_lg,) = _row_call(
        ex_body, E, br,
        [a1b, pxy, lg_attn.reshape(1, hx), bias_lg.reshape(1, hx)],
        ex_in_specs,
        [jax.ShapeDtypeStruct((E, H), f32)],
        [((br, H), lambda i: (i, 0))])

    def exg_body(a_ref, f_ref, attn_ref, bias_ref, o_ref):
        pre = a_ref[:, :hx] + f_ref[:, :hx] + bias_ref[...]
        xh = _leaky(pre).reshape(-1, H, OUT_X)
        attn = attn_ref[...].reshape(1, H, OUT_X)
        o_ref[...] = jnp.exp(jnp.sum(xh * attn, axis=-1))

    (ex_g,) = _row_call(
        exg_body, EG, br,
        [agp, fgm, g_attn.reshape(1, hx), bias_g.reshape(1, hx)],
        ex_in_specs,
        [jax.ShapeDtypeStruct((EG, H), f32)],
        [((br, H), lambda i: (i, 0))])

    s_lg_arr = jax.ops.segment_sum(ex_lg, lg_dst, num_segments=EG)
    cnt_arr = jax.ops.segment_sum(jnp.ones((E,), f32), lg_dst,
                                  num_segments=EG).reshape(EG, 1)
    sg_arr = jax.ops.segment_sum(ex_g, g_dst, num_segments=NG)
    sy_arr = jax.ops.segment_sum(pxy[:, hx:], lg_dst, num_segments=EG)
    syf = None

    def h2_body(hm1_ref, syf_ref, c_ref, agp_ref, fgm_ref, o_ref):
        cnt = jnp.maximum(c_ref[...], 1.0)
        h_lg = hm1_ref[...] + syf_ref[:, hx:] / cnt
        h_g = agp_ref[:, hx:] + fgm_ref[:, hx:]
        o_ref[...] = jnp.concatenate([h_lg, h_g], axis=1)

    if syf is None:
        syf = jnp.concatenate([jnp.zeros((EG, hx), f32), sy_arr], axis=1)
    (h2,) = _row_call(
        h2_body, EG, br,
        [hm1, syf, cnt_arr, agp, fgm],
        [((br, hx), lambda i: (i, 0)), ((br, 2 * hx), lambda i: (i, 0)),
         ((br, 1), lambda i: (i, 0)),
         ((br, 2 * hx), lambda i: (i, 0)), ((br, 2 * hx), lambda i: (i, 0))],
        [jax.ShapeDtypeStruct((EG, 2 * hx), f32)],
        [((br, 2 * hx), lambda i: (i, 0))])

    def sgt_body(s_ref, o_ref):
        o_ref[...] = jnp.pad(s_ref[...], ((0, 0), (0, 128 - H)))

    (sgt,) = _row_call(
        sgt_body, NG, br, [sg_arr],
        [((br, H), lambda i: (i, 0))],
        [jax.ShapeDtypeStruct((NG, 128), f32)],
        [((br, 128), lambda i: (i, 0))])

    asg = _sc_gather([sgt], [g_dst])
    h2g = _sc_gather([h2], [lg_src])

    def wm_body(h_ref, e_ref, o_ref):
        exv = e_ref[...]
        o_ref[...] = jnp.concatenate(
            [h_ref[:, 0:OUT_M] * exv[:, 0:1],
             h_ref[:, OUT_M:hx] * exv[:, 1:2],
             h_ref[:, hx:]], axis=1)

    (wmsg,) = _row_call(
        wm_body, E, br, [h2g, ex_lg],
        [((br, 2 * hx), lambda i: (i, 0)), ((br, H), lambda i: (i, 0))],
        [jax.ShapeDtypeStruct((E, 2 * hx), f32)],
        [((br, 2 * hx), lambda i: (i, 0))])

    nt = jax.ops.segment_sum(wmsg, lg_dst, num_segments=EG)

    def fin_body(nt_ref, sl_ref, exg_ref, asg_ref, o_ref):
        s_lg = jnp.maximum(sl_ref[...], 1e-30)
        numer = nt_ref[:, :hx].reshape(-1, H, OUT_M)
        tg = nt_ref[:, hx:].reshape(-1, H, OUT_M)
        a_g = exg_ref[...] / asg_ref[:, :H]
        h_lg_new = numer / s_lg[:, :, None]
        g_h_new = tg * a_g[:, :, None]
        o_ref[...] = (jnp.sum(_leaky(h_lg_new), axis=1)
                      + jnp.sum(_leaky(g_h_new), axis=1))

    (out,) = _row_call(
        fin_body, EG, br,
        [nt, s_lg_arr, ex_g, asg],
        [((br, 2 * hx), lambda i: (i, 0)), ((br, H), lambda i: (i, 0)),
         ((br, H), lambda i: (i, 0)),
         ((br, 128), lambda i: (i, 0))],
        [jax.ShapeDtypeStruct((EG, OUT_M), f32)],
        [((br, OUT_M), lambda i: (i, 0))])
    return out
